# baseline (speedup 1.0000x reference)
import jax
import jax.numpy as jnp
from jax.experimental import pallas as pl
from jax.experimental.pallas import tpu as pltpu

_N_ACTIONS = 100
_TILE_B = 16384


def _mlp_kernel(x_ref, w1t_ref, b1_ref, w2t_ref, b2_ref, out_ref):
    x = x_ref[...].astype(jnp.bfloat16)
    h = jnp.dot(x, w1t_ref[...], preferred_element_type=jnp.float32)
    h = jnp.maximum(h + b1_ref[...], 0.0).astype(jnp.bfloat16)
    y = jnp.dot(h, w2t_ref[...], preferred_element_type=jnp.float32)
    out_ref[...] = y + b2_ref[...]


def _round_up(n, m):
    return ((n + m - 1) // m) * m


@jax.jit
def _forward(x, w1t_p, b1_p, w2t_p, b2_p):
    B, F = x.shape
    w1t = w1t_p[:F, :].astype(jnp.bfloat16)
    w2t = w2t_p[:, :_N_ACTIONS].astype(jnp.bfloat16)
    b2 = b2_p[:, :_N_ACTIONS]

    tile_b = min(_TILE_B, _round_up(B, 8))
    Bp = _round_up(B, tile_b)
    if Bp != B:
        x = jnp.pad(x, ((0, Bp - B), (0, 0)))

    out = pl.pallas_call(
        _mlp_kernel,
        out_shape=jax.ShapeDtypeStruct((Bp, _N_ACTIONS), jnp.float32),
        grid=(Bp // tile_b,),
        in_specs=[
            pl.BlockSpec((tile_b, F), lambda i: (i, 0)),
            pl.BlockSpec((F, 128), lambda i: (0, 0)),
            pl.BlockSpec((1, 128), lambda i: (0, 0)),
            pl.BlockSpec((128, _N_ACTIONS), lambda i: (0, 0)),
            pl.BlockSpec((1, _N_ACTIONS), lambda i: (0, 0)),
        ],
        out_specs=pl.BlockSpec((tile_b, _N_ACTIONS), lambda i: (i, 0)),
        compiler_params=pltpu.CompilerParams(
            dimension_semantics=("parallel",)),
    )(x, w1t, b1_p, w2t, b2)

    return out[:B] if Bp != B else out


def kernel(x, w1t_p, b1_p, w2t_p, b2_p):
    return _forward(x, w1t_p, b1_p, w2t_p, b2_p)
